# Initial kernel scaffold; baseline (speedup 1.0000x reference)
import jax
import jax.numpy as jnp
from jax.experimental import pallas as pl


def kernel(pos, features, idx, W0, b0, W1, b1, W2, b2):
    raise NotImplementedError("write your pallas kernel here")


# speedup vs baseline: 5.7193x; 5.7193x over previous
import functools

import jax
import jax.numpy as jnp
from jax import lax
from jax.experimental import pallas as pl
from jax.experimental.pallas import tpu as pltpu
from jax.experimental.pallas import tpu_sc as plsc

_K = 32
_TM = 256
_TB = 128
_CHUNK = 128
_DW = 128


def _knn_body(np3_ref, posT_ref, knn_ref):
    np3 = np3_ref[0]
    posT = posT_ref[0]
    tm = np3.shape[0]
    n = posT.shape[1]
    k = knn_ref.shape[-1]

    nx = np3[:, 0:1]
    ny = np3[:, 1:2]
    nz = np3[:, 2:3]
    px = posT[0:1, :]
    py = posT[1:2, :]
    pz = posT[2:3, :]
    q2 = nx * nx + ny * ny + nz * nz
    p2 = px * px + py * py + pz * pz
    cross = lax.dot_general(
        np3.astype(jnp.bfloat16), posT.astype(jnp.bfloat16),
        (((1,), (0,)), ((), ())), preferred_element_type=jnp.float32)
    dist = (q2 + p2) - 2.0 * cross

    iota = lax.broadcasted_iota(jnp.int32, (1, n), 1)
    kiota = lax.broadcasted_iota(jnp.int32, (1, k), 1)
    big = jnp.int32(n)
    inf = jnp.float32(jnp.inf)
    knn0 = jnp.zeros((tm, k), dtype=jnp.int32)

    def body(i, carry):
        d, knn = carry
        m = jnp.min(d, axis=1, keepdims=True)
        cand = jnp.where(d == m, iota, big)
        sel = jnp.min(cand, axis=1, keepdims=True)
        d = jnp.where(iota == sel, inf, d)
        knn = jnp.where(kiota == i, sel, knn)
        return d, knn

    _, knn = lax.fori_loop(0, k, body, (dist, knn0))
    b = pl.program_id(0)
    knn_ref[0] = knn + b * n


def _mlp_body(g_ref, npz_ref, w0_ref, b0_ref, w1_ref, b1_ref, w2_ref, b2_ref,
              out_ref):
    tb = npz_ref.shape[0]
    k = g_ref.shape[0] // tb
    dw = g_ref.shape[1]
    g = g_ref[...]
    npz = npz_ref[...]
    np_rep = jnp.broadcast_to(npz[:, None, :], (tb, k, dw)).reshape(tb * k, dw)
    x = g - np_rep

    dn = (((1,), (0,)), ((), ()))
    hp = jax.lax.Precision.HIGHEST
    h = jnp.maximum(
        lax.dot_general(x, w0_ref[...], dn, precision=hp,
                        preferred_element_type=jnp.float32) + b0_ref[...], 0.0)
    h = jnp.maximum(
        lax.dot_general(h, w1_ref[...], dn, precision=hp,
                        preferred_element_type=jnp.float32) + b1_ref[...], 0.0)
    h = jnp.maximum(
        lax.dot_general(h, w2_ref[...], dn, precision=hp,
                        preferred_element_type=jnp.float32) + b2_ref[...], 0.0)
    co = h.shape[1]
    out_ref[...] = jnp.max(h.reshape(tb, k, co), axis=1)


def _knn_call(new_pos3, posT):
    b, m, _ = new_pos3.shape
    n = posT.shape[2]
    tm = min(_TM, m)
    return pl.pallas_call(
        _knn_body,
        grid=(b, m // tm),
        in_specs=[
            pl.BlockSpec((1, tm, 3), lambda i, j: (i, j, 0)),
            pl.BlockSpec((1, 3, n), lambda i, j: (i, 0, 0)),
        ],
        out_specs=pl.BlockSpec((1, tm, _K), lambda i, j: (i, j, 0)),
        out_shape=jax.ShapeDtypeStruct((b, m, _K), jnp.int32),
    )(new_pos3, posT)


def _mlp_call(g, npz, w0p, w1t, w2t, b0, b1, b2):
    bm = npz.shape[0]
    dw = g.shape[1]
    tb = min(_TB, bm)
    co = w2t.shape[1]
    ch = w1t.shape[0]
    k = g.shape[0] // bm
    return pl.pallas_call(
        _mlp_body,
        grid=(bm // tb,),
        in_specs=[
            pl.BlockSpec((tb * k, dw), lambda i: (i, 0)),
            pl.BlockSpec((tb, dw), lambda i: (i, 0)),
            pl.BlockSpec((dw, ch), lambda i: (0, 0)),
            pl.BlockSpec((1, ch), lambda i: (0, 0)),
            pl.BlockSpec((ch, ch), lambda i: (0, 0)),
            pl.BlockSpec((1, ch), lambda i: (0, 0)),
            pl.BlockSpec((ch, co), lambda i: (0, 0)),
            pl.BlockSpec((1, co), lambda i: (0, 0)),
        ],
        out_specs=pl.BlockSpec((tb, co), lambda i: (i, 0)),
        out_shape=jax.ShapeDtypeStruct((bm, co), jnp.float32),
    )(g, npz, w0p, b0, w1t, b1, w2t, b2)


def _sc_gather(table, gidx):
    r = gidx.shape[0]
    dw = table.shape[1]
    info = plsc.get_sparse_core_info()
    nw = info.num_cores * info.num_subcores
    per_w = r // nw
    iters = per_w // _CHUNK
    mesh = plsc.VectorSubcoreMesh(core_axis_name="c", subcore_axis_name="s")

    @functools.partial(
        pl.kernel,
        out_type=jax.ShapeDtypeStruct((r, dw), jnp.float32),
        mesh=mesh,
        scratch_types=[
            pltpu.VMEM((_CHUNK,), jnp.int32),
            pltpu.VMEM((_CHUNK, dw), jnp.float32),
            pltpu.SemaphoreType.DMA,
        ],
    )
    def gather_k(tab_hbm, idx_hbm, out_hbm, idx_v, rows_v, sem):
        wid = lax.axis_index("s") * info.num_cores + lax.axis_index("c")
        base = wid * per_w

        def body(i, _):
            off = base + i * _CHUNK
            pltpu.sync_copy(idx_hbm.at[pl.ds(off, _CHUNK)], idx_v)
            pltpu.async_copy(tab_hbm.at[idx_v], rows_v, sem).wait()
            pltpu.sync_copy(rows_v, out_hbm.at[pl.ds(off, _CHUNK)])
            return 0

        lax.fori_loop(0, iters, body, 0)

    return gather_k(table, gidx)


def kernel(pos, features, idx, W0, b0, W1, b1, W2, b2):
    b, n, _ = pos.shape
    m = idx.shape[1]
    c = features.shape[2]
    co = W2.shape[0]
    bm = b * m

    pad = jnp.zeros((b, n, _DW - 3 - c), dtype=jnp.float32)
    table = jnp.concatenate([pos, features, pad], axis=-1).reshape(bm // m * n, _DW)

    offs = (jnp.arange(b, dtype=jnp.int32) * n)[:, None]
    gidx1 = (idx.astype(jnp.int32) + offs).reshape(bm)
    np80 = _sc_gather(table, gidx1)
    new_pos = np80[:, :3].reshape(b, m, 3)

    posT = jnp.swapaxes(pos, 1, 2)
    knn = _knn_call(new_pos, posT)

    g = _sc_gather(table, knn.reshape(bm * _K))

    npz = jnp.concatenate(
        [np80[:, :3], jnp.zeros((bm, _DW - 3), dtype=jnp.float32)], axis=1)
    w0p = jnp.pad(W0, ((0, 0), (0, _DW - W0.shape[1]))).T
    w1t = W1.T
    w2t = W2.T
    feats = _mlp_call(g, npz, w0p, w1t, w2t,
                      b0[None, :], b1[None, :], b2[None, :])
    return new_pos, feats.reshape(b, m, co)


# speedup vs baseline: 9.4940x; 1.6600x over previous
import functools

import jax
import jax.numpy as jnp
from jax import lax
from jax.experimental import pallas as pl
from jax.experimental.pallas import tpu as pltpu
from jax.experimental.pallas import tpu_sc as plsc

_K = 32
_TM = 128
_TB = 128
_CHUNK = 128
_DW = 128


def _knn_body(np3_ref, posT_ref, knn_ref, dscr):
    np3 = np3_ref[0]
    posT = posT_ref[0]
    tm = np3.shape[0]
    n = posT.shape[1]
    k = knn_ref.shape[-1]
    nt = n // 128

    nx = np3[:, 0:1]
    ny = np3[:, 1:2]
    nz = np3[:, 2:3]
    px = posT[0:1, :]
    py = posT[1:2, :]
    pz = posT[2:3, :]
    q2 = nx * nx + ny * ny + nz * nz
    p2 = px * px + py * py + pz * pz
    cross = lax.dot_general(
        np3.astype(jnp.bfloat16), posT.astype(jnp.bfloat16),
        (((1,), (0,)), ((), ())), preferred_element_type=jnp.float32)
    dscr[...] = (q2 + p2) - 2.0 * cross

    li = lax.broadcasted_iota(jnp.int32, (1, 128), 1)
    kiota = lax.broadcasted_iota(jnp.int32, (1, k), 1)
    big = jnp.int32(n)
    inf = jnp.float32(jnp.inf)
    knn0 = jnp.zeros((tm, k), dtype=jnp.int32)

    def body(i, carry):
        sel_prev, knn = carry
        plval = jnp.full((tm, 128), inf, jnp.float32)
        plarg = jnp.zeros((tm, 128), jnp.int32)
        for c in range(nt):
            t = dscr[:, c * 128:(c + 1) * 128]
            t = jnp.where(li == sel_prev - c * 128, inf, t)
            dscr[:, c * 128:(c + 1) * 128] = t
            lt = t < plval
            plval = jnp.where(lt, t, plval)
            plarg = jnp.where(lt, jnp.int32(c), plarg)
        m = jnp.min(plval, axis=1, keepdims=True)
        gidxs = jnp.where(plval == m, plarg * 128 + li, big)
        sel = jnp.min(gidxs, axis=1, keepdims=True)
        knn = jnp.where(kiota == i, sel, knn)
        return sel, knn

    sel0 = jnp.full((tm, 1), -1, jnp.int32)
    _, knn = lax.fori_loop(0, k, body, (sel0, knn0))
    b = pl.program_id(0)
    knn_ref[0] = knn + b * n


def _mlp_body(g_ref, npz_ref, w0_ref, b0_ref, w1_ref, b1_ref, w2_ref, b2_ref,
              out_ref):
    tb = npz_ref.shape[0]
    k = g_ref.shape[0] // tb
    dw = g_ref.shape[1]
    g = g_ref[...]
    npz = npz_ref[...]
    np_rep = jnp.broadcast_to(npz[:, None, :], (tb, k, dw)).reshape(tb * k, dw)
    x = g - np_rep

    dn = (((1,), (0,)), ((), ()))
    hp = jax.lax.Precision.DEFAULT
    h = jnp.maximum(
        lax.dot_general(x, w0_ref[...], dn, precision=hp,
                        preferred_element_type=jnp.float32) + b0_ref[...], 0.0)
    h = jnp.maximum(
        lax.dot_general(h, w1_ref[...], dn, precision=hp,
                        preferred_element_type=jnp.float32) + b1_ref[...], 0.0)
    h = jnp.maximum(
        lax.dot_general(h, w2_ref[...], dn, precision=hp,
                        preferred_element_type=jnp.float32) + b2_ref[...], 0.0)
    co = h.shape[1]
    out_ref[...] = jnp.max(h.reshape(tb, k, co), axis=1)


def _knn_call(new_pos3, posT):
    b, m, _ = new_pos3.shape
    n = posT.shape[2]
    tm = min(_TM, m)
    return pl.pallas_call(
        _knn_body,
        grid=(b, m // tm),
        in_specs=[
            pl.BlockSpec((1, tm, 3), lambda i, j: (i, j, 0)),
            pl.BlockSpec((1, 3, n), lambda i, j: (i, 0, 0)),
        ],
        out_specs=pl.BlockSpec((1, tm, _K), lambda i, j: (i, j, 0)),
        out_shape=jax.ShapeDtypeStruct((b, m, _K), jnp.int32),
        scratch_shapes=[pltpu.VMEM((tm, n), jnp.float32)],
    )(new_pos3, posT)


def _mlp_call(g, npz, w0p, w1t, w2t, b0, b1, b2):
    bm = npz.shape[0]
    dw = g.shape[1]
    tb = min(_TB, bm)
    co = w2t.shape[1]
    ch = w1t.shape[0]
    k = g.shape[0] // bm
    return pl.pallas_call(
        _mlp_body,
        grid=(bm // tb,),
        in_specs=[
            pl.BlockSpec((tb * k, dw), lambda i: (i, 0)),
            pl.BlockSpec((tb, dw), lambda i: (i, 0)),
            pl.BlockSpec((dw, ch), lambda i: (0, 0)),
            pl.BlockSpec((1, ch), lambda i: (0, 0)),
            pl.BlockSpec((ch, ch), lambda i: (0, 0)),
            pl.BlockSpec((1, ch), lambda i: (0, 0)),
            pl.BlockSpec((ch, co), lambda i: (0, 0)),
            pl.BlockSpec((1, co), lambda i: (0, 0)),
        ],
        out_specs=pl.BlockSpec((tb, co), lambda i: (i, 0)),
        out_shape=jax.ShapeDtypeStruct((bm, co), jnp.float32),
    )(g, npz, w0p, b0, w1t, b1, w2t, b2)


def _sc_gather(table, gidx):
    r = gidx.shape[0]
    dw = table.shape[1]
    info = plsc.get_sparse_core_info()
    nw = info.num_cores * info.num_subcores
    per_w = r // nw
    iters = per_w // _CHUNK
    mesh = plsc.VectorSubcoreMesh(core_axis_name="c", subcore_axis_name="s")

    @functools.partial(
        pl.kernel,
        out_type=jax.ShapeDtypeStruct((r, dw), jnp.float32),
        mesh=mesh,
        scratch_types=[
            pltpu.VMEM((_CHUNK,), jnp.int32),
            pltpu.VMEM((_CHUNK, dw), jnp.float32),
            pltpu.SemaphoreType.DMA,
        ],
    )
    def gather_k(tab_hbm, idx_hbm, out_hbm, idx_v, rows_v, sem):
        wid = lax.axis_index("s") * info.num_cores + lax.axis_index("c")
        base = wid * per_w

        def body(i, _):
            off = base + i * _CHUNK
            pltpu.sync_copy(idx_hbm.at[pl.ds(off, _CHUNK)], idx_v)
            pltpu.async_copy(tab_hbm.at[idx_v], rows_v, sem).wait()
            pltpu.sync_copy(rows_v, out_hbm.at[pl.ds(off, _CHUNK)])
            return 0

        lax.fori_loop(0, iters, body, 0)

    return gather_k(table, gidx)


def kernel(pos, features, idx, W0, b0, W1, b1, W2, b2):
    b, n, _ = pos.shape
    m = idx.shape[1]
    c = features.shape[2]
    co = W2.shape[0]
    bm = b * m

    pad = jnp.zeros((b, n, _DW - 3 - c), dtype=jnp.float32)
    table = jnp.concatenate([pos, features, pad], axis=-1).reshape(bm // m * n, _DW)

    offs = (jnp.arange(b, dtype=jnp.int32) * n)[:, None]
    gidx1 = (idx.astype(jnp.int32) + offs).reshape(bm)
    np80 = _sc_gather(table, gidx1)
    new_pos = np80[:, :3].reshape(b, m, 3)

    posT = jnp.swapaxes(pos, 1, 2)
    npz = jnp.concatenate(
        [np80[:, :3], jnp.zeros((bm, _DW - 3), dtype=jnp.float32)], axis=1)
    w0p = jnp.pad(W0, ((0, 0), (0, _DW - W0.shape[1]))).T
    w1t = W1.T
    w2t = W2.T
    b0r, b1r, b2r = b0[None, :], b1[None, :], b2[None, :]

    feats_parts = []
    for i in range(b):
        knn_i = _knn_call(new_pos[i:i + 1], posT[i:i + 1])
        g_i = _sc_gather(table, (knn_i + i * n).reshape(m * _K))
        feats_parts.append(
            _mlp_call(g_i, npz[i * m:(i + 1) * m], w0p, w1t, w2t,
                      b0r, b1r, b2r))
    feats = jnp.concatenate(feats_parts, axis=0)
    return new_pos, feats.reshape(b, m, co)
